# Initial kernel scaffold; baseline (speedup 1.0000x reference)
import jax
import jax.numpy as jnp
from jax.experimental import pallas as pl


def kernel(feature, edge_index, edge_index2, norm_A, norm_A_2, W1, b1, W2, b2, temp1, temp2):
    raise NotImplementedError("write your pallas kernel here")


# speedup vs baseline: 1.7595x; 1.7595x over previous
import functools

import jax
import jax.numpy as jnp
from jax import lax
from jax.experimental import pallas as pl
from jax.experimental.pallas import tpu as pltpu
from jax.experimental.pallas import tpu_sc as plsc

N = 10000
NPAD = 10240
C = 64
CW = 128
K = 10
NTILES = 16
ROWS = NPAD // NTILES
E = 320000
EPT = E // NTILES
W = 128
EPT_PAD = 20480
NCHUNK = EPT_PAD // W
NRC = ROWS // W


def _mlp_body(f_ref, w1_ref, b1_ref, w2_ref, b2_ref, o_ref):
    h = jnp.dot(f_ref[...], w1_ref[...],
                preferred_element_type=jnp.float32) + b1_ref[...]
    h = jnp.maximum(h, 0.0)
    x = jnp.dot(h, w2_ref[...],
                preferred_element_type=jnp.float32) + b2_ref[...]
    o_ref[...] = jnp.concatenate(
        [x, jnp.zeros((x.shape[0], CW - C), jnp.float32)], axis=1)


def _mlp(feature_pad, W1, b1, W2, b2):
    nblk = NPAD // 1024
    return pl.pallas_call(
        _mlp_body,
        grid=(nblk,),
        in_specs=[
            pl.BlockSpec((1024, 128), lambda i: (i, 0)),
            pl.BlockSpec((128, 128), lambda i: (0, 0)),
            pl.BlockSpec((1, 128), lambda i: (0, 0)),
            pl.BlockSpec((128, C), lambda i: (0, 0)),
            pl.BlockSpec((1, C), lambda i: (0, 0)),
        ],
        out_specs=pl.BlockSpec((1024, CW), lambda i: (i, 0)),
        out_shape=jax.ShapeDtypeStruct((NPAD, CW), jnp.float32),
    )(feature_pad, W1, b1.reshape(1, 128), W2, b2.reshape(1, C))


def _add_body(a_ref, b_ref, o_ref):
    o_ref[...] = a_ref[...] + b_ref[...]


def _final_add(a, b):
    return pl.pallas_call(
        _add_body,
        grid=(10,),
        in_specs=[
            pl.BlockSpec((1000, C), lambda i: (i, 0)),
            pl.BlockSpec((1000, C), lambda i: (i, 0)),
        ],
        out_specs=pl.BlockSpec((1000, C), lambda i: (i, 0)),
        out_shape=jax.ShapeDtypeStruct((N, C), jnp.float32),
    )(a, b)


def _splat(vec16, idx):
    idxs = jnp.broadcast_to(jnp.asarray(idx, jnp.int32), (16,))
    return jnp.take_along_axis(vec16, idxs, axis=0)


def _prop_body(xw_hbm, src_hbm, dst_hbm, norm_hbm, temps_hbm,
               out_hbm, hw_hbm,
               hnext, srcb, dstb, normb, msg, obuf, temps_v):
    cid = lax.axis_index("c")
    sid = lax.axis_index("s")
    rbase = sid * ROWS

    pltpu.sync_copy(temps_hbm.at[cid], temps_v)
    t16 = temps_v[...]
    t0 = _splat(t16, 0)

    for i in range(NRC):
        rs = pl.ds(rbase + i * W, W)
        pltpu.sync_copy(xw_hbm.at[rs], msg)

        def _s0(r, _):
            for j in range(C // 16):
                sl = pl.ds(j * 16, 16)
                v = msg[r, sl] * t0
                msg[r, sl] = v
                obuf[r, sl] = v
            return 0
        lax.fori_loop(0, W, _s0, 0)
        pltpu.sync_copy(msg, hw_hbm.at[cid, 1, rs])
        pltpu.sync_copy(obuf, out_hbm.at[cid, rs])

    def _hop(bsrc, bdst, k):
        def _z(r, _):
            for j in range(CW // 16):
                msg[r, pl.ds(j * 16, 16)] = jnp.zeros((16,), jnp.float32)
            return 0
        lax.fori_loop(0, W, _z, 0)
        for i in range(NRC):
            pltpu.sync_copy(msg, hnext.at[pl.ds(rbase + i * W, W)])
        plsc.subcore_barrier()

        rk = _splat(t16, k + 1) / _splat(t16, k)
        hsrc = hw_hbm.at[cid, bsrc]

        def _chunk(c, _):
            pltpu.sync_copy(src_hbm.at[cid, sid, c], srcb.at[0])
            pltpu.sync_copy(dst_hbm.at[cid, sid, c], dstb.at[0])
            pltpu.sync_copy(norm_hbm.at[cid, sid, c], normb.at[0])
            pltpu.sync_copy(hsrc.at[srcb.at[0]], msg)

            def _q(q, _):
                n16 = normb[0, pl.ds(q * 16, 16)] * rk
                for e in range(16):
                    nb = jnp.broadcast_to(
                        lax.slice(n16, (e,), (e + 1,)), (16,))
                    row = q * 16 + e
                    for j in range(C // 16):
                        sl = pl.ds(j * 16, 16)
                        msg[row, sl] = msg[row, sl] * nb
                return 0
            lax.fori_loop(0, W // 16, _q, 0)
            pltpu.sync_copy(msg, hnext.at[dstb.at[0]], add=True)
            return 0
        lax.fori_loop(0, NCHUNK, _chunk, 0)
        plsc.subcore_barrier()

        for i in range(NRC):
            rs = pl.ds(rbase + i * W, W)
            pltpu.sync_copy(hnext.at[rs], msg)
            pltpu.sync_copy(msg, hw_hbm.at[cid, bdst, rs])
            pltpu.sync_copy(out_hbm.at[cid, rs], obuf)

            def _a(r, _):
                for j in range(C // 16):
                    sl = pl.ds(j * 16, 16)
                    obuf[r, sl] = obuf[r, sl] + msg[r, sl]
                return 0
            lax.fori_loop(0, W, _a, 0)
            pltpu.sync_copy(obuf, out_hbm.at[cid, rs])

    def _pair(it, _):
        _hop(1, 0, 2 * it)
        _hop(0, 1, 2 * it + 1)
        return 0
    lax.fori_loop(0, K // 2, _pair, 0)


@functools.lru_cache(maxsize=None)
def _get_prop():
  return pl.kernel(
    _prop_body,
    out_type=(
        jax.ShapeDtypeStruct((2, NPAD, C), jnp.float32),
        jax.ShapeDtypeStruct((2, 2, NPAD, CW), jnp.float32),
    ),
    mesh=plsc.VectorSubcoreMesh(core_axis_name="c", subcore_axis_name="s",
                                num_cores=2, num_subcores=NTILES),
    scratch_types=[
        pltpu.VMEM_SHARED((NPAD, CW), jnp.float32),
        pltpu.VMEM((2, W), jnp.int32),
        pltpu.VMEM((2, W), jnp.int32),
        pltpu.VMEM((2, W), jnp.float32),
        pltpu.VMEM((W, CW), jnp.float32),
        pltpu.VMEM((W, C), jnp.float32),
        pltpu.VMEM((16,), jnp.float32),
    ],
  )


def _prep_edges(ei, nrm):
    src = ei[0].astype(jnp.int32).reshape(NTILES, EPT)
    dst = ei[1].astype(jnp.int32).reshape(NTILES, EPT)
    nr = nrm.astype(jnp.float32).reshape(NTILES, EPT)
    pad = EPT_PAD - EPT
    psrc = jnp.full((NTILES, pad), N - 2, jnp.int32)
    pdst = jnp.broadcast_to(N + (jnp.arange(pad, dtype=jnp.int32) % 240),
                            (NTILES, pad))
    pnrm = jnp.zeros((NTILES, pad), jnp.float32)
    src = jnp.concatenate([src, psrc], 1).reshape(NTILES, NCHUNK, W)
    dst = jnp.concatenate([dst, pdst], 1).reshape(NTILES, NCHUNK, W)
    nr = jnp.concatenate([nr, pnrm], 1).reshape(NTILES, NCHUNK, W)
    return src, dst, nr


def kernel(feature, edge_index, edge_index2, norm_A, norm_A_2,
           W1, b1, W2, b2, temp1, temp2):
    feature_pad = jnp.pad(feature, ((0, NPAD - N), (0, 0)))
    x = _mlp(feature_pad, W1, b1, W2, b2)

    s1, d1, n1 = _prep_edges(edge_index, norm_A)
    s2, d2, n2 = _prep_edges(edge_index2, norm_A_2)
    srcs = jnp.stack([s1, s2])
    dsts = jnp.stack([d1, d2])
    norms = jnp.stack([n1, n2])
    temps = jnp.stack([jnp.pad(temp1, (0, 5)), jnp.pad(temp2, (0, 5))])

    out, _ = _get_prop()(x, srcs, dsts, norms, temps)
    return _final_add(out[0, :N], out[1, :N])


# speedup vs baseline: 2.4637x; 1.4002x over previous
import functools

import jax
import jax.numpy as jnp
from jax import lax
from jax.experimental import pallas as pl
from jax.experimental.pallas import tpu as pltpu
from jax.experimental.pallas import tpu_sc as plsc

N = 10000
NPAD = 10240
C = 64
CW = 128
K = 10
NTILES = 16
ROWS = NPAD // NTILES
E = 320000
EPT = E // NTILES
W = 128
EPT_PAD = 20480
NCHUNK = EPT_PAD // W
NRC = ROWS // W


def _mlp_body(f_ref, w1_ref, b1_ref, w2_ref, b2_ref, o_ref):
    h = jnp.dot(f_ref[...], w1_ref[...],
                preferred_element_type=jnp.float32) + b1_ref[...]
    h = jnp.maximum(h, 0.0)
    x = jnp.dot(h, w2_ref[...],
                preferred_element_type=jnp.float32) + b2_ref[...]
    o_ref[...] = jnp.concatenate(
        [x, jnp.zeros((x.shape[0], CW - C), jnp.float32)], axis=1)


def _mlp(feature_pad, W1, b1, W2, b2):
    nblk = NPAD // 1024
    return pl.pallas_call(
        _mlp_body,
        grid=(nblk,),
        in_specs=[
            pl.BlockSpec((1024, 128), lambda i: (i, 0)),
            pl.BlockSpec((128, 128), lambda i: (0, 0)),
            pl.BlockSpec((1, 128), lambda i: (0, 0)),
            pl.BlockSpec((128, C), lambda i: (0, 0)),
            pl.BlockSpec((1, C), lambda i: (0, 0)),
        ],
        out_specs=pl.BlockSpec((1024, CW), lambda i: (i, 0)),
        out_shape=jax.ShapeDtypeStruct((NPAD, CW), jnp.float32),
    )(feature_pad, W1, b1.reshape(1, 128), W2, b2.reshape(1, C))


def _add_body(a_ref, b_ref, o_ref):
    o_ref[...] = a_ref[...] + b_ref[...]


def _final_add(a, b):
    return pl.pallas_call(
        _add_body,
        grid=(10,),
        in_specs=[
            pl.BlockSpec((1000, C), lambda i: (i, 0)),
            pl.BlockSpec((1000, C), lambda i: (i, 0)),
        ],
        out_specs=pl.BlockSpec((1000, C), lambda i: (i, 0)),
        out_shape=jax.ShapeDtypeStruct((N, C), jnp.float32),
    )(a, b)


def _splat(vec16, idx):
    idxs = jnp.broadcast_to(jnp.asarray(idx, jnp.int32), (16,))
    return jnp.take_along_axis(vec16, idxs, axis=0)


def _prop_body(xw_hbm, edges_hbm, temps_hbm,
               out_hbm, hw_hbm,
               hnext, ebufA, ebufB, msgA, msgB, temps_v,
               semGA, semGB):
    cid = lax.axis_index("c")
    sid = lax.axis_index("s")
    rbase = sid * ROWS

    pltpu.sync_copy(temps_hbm.at[cid], temps_v)
    t16 = temps_v[...]
    t0 = _splat(t16, 0)

    for i in range(NRC):
        rs = pl.ds(rbase + i * W, W)
        pltpu.sync_copy(xw_hbm.at[rs], msgA)

        def _s0(r, _):
            for j in range(C // 16):
                sl = pl.ds(j * 16, 16)
                msgA[r, sl] = msgA[r, sl] * t0
            return 0
        lax.fori_loop(0, W, _s0, 0)
        pltpu.sync_copy(msgA, hw_hbm.at[cid, 1, rs])
        pltpu.sync_copy(msgA, out_hbm.at[cid, rs])

    def _scale(mref, eref, rk):
        def _q(q, _):
            n16 = lax.bitcast_convert_type(
                eref[2, pl.ds(q * 16, 16)], jnp.float32) * rk
            for e in range(16):
                nb = jnp.broadcast_to(
                    lax.slice(n16, (e,), (e + 1,)), (16,))
                row = q * 16 + e
                for j in range(C // 16):
                    sl = pl.ds(j * 16, 16)
                    mref[row, sl] = mref[row, sl] * nb
            return 0
        lax.fori_loop(0, W // 16, _q, 0)

    def _hop(bsrc, bdst, k):
        def _z(r, _):
            for j in range(CW // 16):
                msgA[r, pl.ds(j * 16, 16)] = jnp.zeros((16,), jnp.float32)
            return 0
        lax.fori_loop(0, W, _z, 0)
        for i in range(NRC):
            pltpu.sync_copy(msgA, hnext.at[pl.ds(rbase + i * W, W)])
        plsc.subcore_barrier()

        rk = _splat(t16, k + 1) / _splat(t16, k)
        hsrc = hw_hbm.at[cid, bsrc]

        pltpu.sync_copy(edges_hbm.at[cid, sid, 0], ebufA)
        pltpu.async_copy(hsrc.at[ebufA.at[0]], msgA, semGA)

        def _pair(m, _):
            c0 = 2 * m
            pltpu.sync_copy(edges_hbm.at[cid, sid, c0 + 1], ebufB)
            pltpu.async_copy(hsrc.at[ebufB.at[0]], msgB, semGB)
            pltpu.make_async_copy(hsrc.at[ebufA.at[0]], msgA, semGA).wait()
            _scale(msgA, ebufA, rk)
            pltpu.sync_copy(msgA, hnext.at[ebufA.at[1]], add=True)
            c2 = jnp.minimum(c0 + 2, NCHUNK - 1)
            pltpu.sync_copy(edges_hbm.at[cid, sid, c2], ebufA)
            pltpu.async_copy(hsrc.at[ebufA.at[0]], msgA, semGA)
            pltpu.make_async_copy(hsrc.at[ebufB.at[0]], msgB, semGB).wait()
            _scale(msgB, ebufB, rk)
            pltpu.sync_copy(msgB, hnext.at[ebufB.at[1]], add=True)
            return 0
        lax.fori_loop(0, NCHUNK // 2, _pair, 0)
        pltpu.make_async_copy(hsrc.at[ebufA.at[0]], msgA, semGA).wait()
        plsc.subcore_barrier()

        for i in range(NRC):
            rs = pl.ds(rbase + i * W, W)
            pltpu.sync_copy(hnext.at[rs], msgA)
            pltpu.sync_copy(msgA, hw_hbm.at[cid, bdst, rs])
            pltpu.sync_copy(out_hbm.at[cid, rs], msgB)

            def _a(r, _):
                for j in range(C // 16):
                    sl = pl.ds(j * 16, 16)
                    msgB[r, sl] = msgB[r, sl] + msgA[r, sl]
                return 0
            lax.fori_loop(0, W, _a, 0)
            pltpu.sync_copy(msgB, out_hbm.at[cid, rs])

    def _pair(it, _):
        _hop(1, 0, 2 * it)
        _hop(0, 1, 2 * it + 1)
        return 0
    lax.fori_loop(0, K // 2, _pair, 0)


@functools.lru_cache(maxsize=None)
def _get_prop():
  return pl.kernel(
    _prop_body,
    out_type=(
        jax.ShapeDtypeStruct((2, NPAD, CW), jnp.float32),
        jax.ShapeDtypeStruct((2, 2, NPAD, CW), jnp.float32),
    ),
    mesh=plsc.VectorSubcoreMesh(core_axis_name="c", subcore_axis_name="s",
                                num_cores=2, num_subcores=NTILES),
    scratch_types=[
        pltpu.VMEM_SHARED((NPAD, CW), jnp.float32),
        pltpu.VMEM((3, W), jnp.int32),
        pltpu.VMEM((3, W), jnp.int32),
        pltpu.VMEM((W, CW), jnp.float32),
        pltpu.VMEM((W, CW), jnp.float32),
        pltpu.VMEM((16,), jnp.float32),
        pltpu.SemaphoreType.DMA,
        pltpu.SemaphoreType.DMA,
    ],
  )


def _prep_edges(ei, nrm):
    src = ei[0].astype(jnp.int32).reshape(NTILES, EPT)
    dst = ei[1].astype(jnp.int32).reshape(NTILES, EPT)
    nr = nrm.astype(jnp.float32).reshape(NTILES, EPT)
    pad = EPT_PAD - EPT
    psrc = jnp.full((NTILES, pad), N - 2, jnp.int32)
    pdst = jnp.broadcast_to(N + (jnp.arange(pad, dtype=jnp.int32) % 240),
                            (NTILES, pad))
    pnrm = jnp.zeros((NTILES, pad), jnp.float32)
    src = jnp.concatenate([src, psrc], 1).reshape(NTILES, NCHUNK, W)
    dst = jnp.concatenate([dst, pdst], 1).reshape(NTILES, NCHUNK, W)
    nr = lax.bitcast_convert_type(
        jnp.concatenate([nr, pnrm], 1).reshape(NTILES, NCHUNK, W), jnp.int32)
    return jnp.stack([src, dst, nr], axis=2)


def kernel(feature, edge_index, edge_index2, norm_A, norm_A_2,
           W1, b1, W2, b2, temp1, temp2):
    feature_pad = jnp.pad(feature, ((0, NPAD - N), (0, 0)))
    x = _mlp(feature_pad, W1, b1, W2, b2)

    e1 = _prep_edges(edge_index, norm_A)
    e2 = _prep_edges(edge_index2, norm_A_2)
    edges = jnp.stack([e1, e2])
    temps = jnp.stack([jnp.pad(temp1, (0, 5)), jnp.pad(temp2, (0, 5))])

    out, _ = _get_prop()(x, edges, temps)
    return _final_add(out[0, :N, :C], out[1, :N, :C])
